# baseline (speedup 1.0000x reference)
import jax
import jax.numpy as jnp
from jax import lax
from jax.experimental import pallas as pl
from jax.experimental.pallas import tpu as pltpu
from jax.experimental.pallas import tpu_sc as plsc

_NC = 2
_NS = 16
_NW = _NC * _NS
_L = 16
_G = 32


def _make_pool(B, N, C, K):
    rpw = (B * K) // _NW
    c2 = C // 2
    cl = c2 // _L

    def body(xs_hbm, ys_hbm, zs_hbm, feats_hbm, props_hbm, out_hbm,
             xs_v, ys_v, zs_v, props_v, meml, rows_a, rows_b, outrow_v,
             sem_a, sem_b):
        wid = lax.axis_index("s") * _NC + lax.axis_index("c")
        g0 = wid * rpw
        base = (g0 // K) * N
        pltpu.sync_copy(xs_hbm.at[pl.ds(base, N)], xs_v)
        pltpu.sync_copy(ys_hbm.at[pl.ds(base, N)], ys_v)
        pltpu.sync_copy(zs_hbm.at[pl.ds(base, N)], zs_v)
        pltpu.sync_copy(props_hbm.at[pl.ds(g0 * _L, rpw * _L)], props_v)

        iota = lax.iota(jnp.int32, _L)

        def roi_body(r, carry):
            prow = props_v[pl.ds(r * _L, _L)]
            lox, hix, loy, hiy, loz, hiz = (prow[0], prow[1], prow[2],
                                            prow[3], prow[4], prow[5])

            def chunk(i, off):
                x = xs_v[pl.ds(i * _L, _L)]
                y = ys_v[pl.ds(i * _L, _L)]
                z = zs_v[pl.ds(i * _L, _L)]
                m = ((x > lox) & (x < hix) & (y > loy) & (y < hiy)
                     & (z > loz) & (z < hiz))
                idxv = (base + i * _L) + iota
                plsc.store_compressed(meml.at[pl.ds(off, _L)], idxv, mask=m)
                return off + plsc.all_reduce_population_count(m)[0]

            def mask_body(i2, off):
                off = chunk(2 * i2, off)
                return chunk(2 * i2 + 1, off)

            cnt = lax.fori_loop(0, N // (2 * _L), mask_body, jnp.int32(0))

            mv = meml[pl.ds(0, _L)]
            padv = mv[0] + (iota * 0)
            for q in range(2 * _G // _L):
                plsc.store_scatter(meml, [cnt + q * _L + iota], padv)
            npair = (cnt + (2 * _G - 1)) // (2 * _G)
            nch = 2 * npair

            def start(j, rows, sem):
                pltpu.async_copy(
                    feats_hbm.at[meml.at[pl.ds(j * _G, _G)]], rows, sem)

            def wait(j, rows, sem):
                pltpu.make_async_copy(
                    feats_hbm.at[meml.at[pl.ds(j * _G, _G)]],
                    rows, sem).wait()

            def accum(rows, acc):
                def row_body(t, a):
                    return tuple(
                        jnp.maximum(a[c], plsc.bitcast(
                            rows[t, pl.ds(c * _L, _L)], jnp.bfloat16))
                        for c in range(cl))
                return lax.fori_loop(0, _G, row_body, tuple(acc))

            @pl.when(npair > 0)
            def _():
                start(0, rows_a, sem_a)

            def pair_body(p, acc):
                j0 = 2 * p
                start(j0 + 1, rows_b, sem_b)
                wait(j0, rows_a, sem_a)
                acc = accum(rows_a, acc)

                @pl.when(j0 + 2 < nch)
                def _():
                    start(j0 + 2, rows_a, sem_a)

                wait(j0 + 1, rows_b, sem_b)
                return accum(rows_b, acc)

            acc0 = tuple(jnp.full((2 * _L,), -jnp.inf, jnp.bfloat16)
                         for _ in range(cl))
            acc = lax.fori_loop(0, npair, pair_body, acc0)
            nonempty = cnt > 0
            for c in range(cl):
                outrow_v[pl.ds(c * _L, _L)] = plsc.bitcast(
                    jnp.where(nonempty, acc[c], jnp.bfloat16(0.0)),
                    jnp.float32)
            pltpu.sync_copy(outrow_v, out_hbm.at[g0 + r])
            return carry

        lax.fori_loop(0, rpw, roi_body, jnp.int32(0))

    mesh = plsc.VectorSubcoreMesh(core_axis_name="c", subcore_axis_name="s",
                                  num_cores=_NC, num_subcores=_NS)
    return pl.kernel(
        body,
        out_type=jax.ShapeDtypeStruct((B * K, c2), jnp.float32),
        mesh=mesh,
        compiler_params=pltpu.CompilerParams(
            needs_layout_passes=False,
            use_tc_tiling_on_sc=False,
        ),
        scratch_types=[
            pltpu.VMEM((N,), jnp.float32),
            pltpu.VMEM((N,), jnp.float32),
            pltpu.VMEM((N,), jnp.float32),
            pltpu.VMEM((rpw * _L,), jnp.float32),
            pltpu.VMEM((N + 2 * _G,), jnp.int32),
            pltpu.VMEM((_G, c2), jnp.float32),
            pltpu.VMEM((_G, c2), jnp.float32),
            pltpu.VMEM((c2,), jnp.float32),
            pltpu.SemaphoreType.DMA,
            pltpu.SemaphoreType.DMA,
        ],
    )


def _fc_body(p_ref, w1_ref, b1_ref, w2_ref, b2_ref, o_ref):
    h = jnp.dot(p_ref[...], w1_ref[...],
                preferred_element_type=jnp.float32) + b1_ref[...]
    h = jnp.maximum(h, 0.0)
    o = jnp.dot(h, w2_ref[...],
                preferred_element_type=jnp.float32) + b2_ref[...]
    o_ref[...] = jnp.maximum(o, 0.0)


def kernel(points, point_features, proposals, W1, b1, W2, b2):
    B, N, C = point_features.shape
    K = proposals.shape[1]

    xs = points[..., 0].reshape(B * N)
    ys = points[..., 1].reshape(B * N)
    zs = points[..., 2].reshape(B * N)
    feats_flat = jax.lax.bitcast_convert_type(
        point_features.astype(jnp.bfloat16).reshape(B * N, C // 2, 2),
        jnp.float32)
    ctr = proposals[..., 0:3]
    half = proposals[..., 3:6] / 2
    lo = ctr - half
    hi = ctr + half
    props = jnp.stack([lo[..., 0], hi[..., 0], lo[..., 1], hi[..., 1],
                       lo[..., 2], hi[..., 2]], axis=-1)
    props = jnp.concatenate(
        [props, jnp.zeros((B, K, _L - 6), jnp.float32)],
        axis=-1).reshape(B * K * _L)

    pooled = _make_pool(B, N, C, K)(xs, ys, zs, feats_flat, props)
    pooled = jax.lax.bitcast_convert_type(
        pooled, jnp.bfloat16).reshape(B * K, C).astype(jnp.float32)

    out = pl.pallas_call(
        _fc_body,
        out_shape=jax.ShapeDtypeStruct((B * K, W2.shape[1]), jnp.float32),
    )(pooled, W1, b1.reshape(1, -1), W2, b2.reshape(1, -1))
    return out.reshape(B, K, W2.shape[1])


# speedup vs baseline: 1.1453x; 1.1453x over previous
import jax
import jax.numpy as jnp
from jax import lax
from jax.experimental import pallas as pl
from jax.experimental.pallas import tpu as pltpu
from jax.experimental.pallas import tpu_sc as plsc

_NC = 2
_NS = 16
_NW = _NC * _NS
_L = 16
_G = 64


def _make_pool(B, N, C, K):
    rpw = (B * K) // _NW
    cl = C // _L

    def body(xs_hbm, ys_hbm, zs_hbm, feats_hbm, props_hbm, out_hbm,
             xs_v, ys_v, zs_v, props_v, meml, rows_a, rows_b, outrow_v,
             sem_a, sem_b):
        wid = lax.axis_index("s") * _NC + lax.axis_index("c")
        g0 = wid * rpw
        base = (g0 // K) * N
        pltpu.sync_copy(xs_hbm.at[pl.ds(base, N)], xs_v)
        pltpu.sync_copy(ys_hbm.at[pl.ds(base, N)], ys_v)
        pltpu.sync_copy(zs_hbm.at[pl.ds(base, N)], zs_v)
        pltpu.sync_copy(props_hbm.at[pl.ds(g0 * _L, rpw * _L)], props_v)

        iota = lax.iota(jnp.int32, _L)

        def roi_body(r, carry):
            prow = props_v[pl.ds(r * _L, _L)]
            lox, hix, loy, hiy, loz, hiz = (prow[0], prow[1], prow[2],
                                            prow[3], prow[4], prow[5])

            def chunk(i, off):
                x = xs_v[pl.ds(i * _L, _L)]
                y = ys_v[pl.ds(i * _L, _L)]
                z = zs_v[pl.ds(i * _L, _L)]
                m = ((x > lox) & (x < hix) & (y > loy) & (y < hiy)
                     & (z > loz) & (z < hiz))
                idxv = (base + i * _L) + iota
                plsc.store_compressed(meml.at[pl.ds(off, _L)], idxv, mask=m)
                return off + plsc.all_reduce_population_count(m)[0]

            def mask_body(i2, off):
                off = chunk(2 * i2, off)
                return chunk(2 * i2 + 1, off)

            cnt = lax.fori_loop(0, N // (2 * _L), mask_body, jnp.int32(0))

            mv = meml[pl.ds(0, _L)]
            padv = mv[0] + (iota * 0)
            for q in range(2 * _G // _L):
                plsc.store_scatter(meml, [cnt + q * _L + iota], padv)
            npair = (cnt + (2 * _G - 1)) // (2 * _G)
            nch = 2 * npair

            def start(j, rows, sem):
                pltpu.async_copy(
                    feats_hbm.at[meml.at[pl.ds(j * _G, _G)]], rows, sem)

            def wait(j, rows, sem):
                pltpu.make_async_copy(
                    feats_hbm.at[meml.at[pl.ds(j * _G, _G)]],
                    rows, sem).wait()

            def accum(rows, acc):
                def row_body(t, a):
                    return tuple(
                        jnp.maximum(a[c], rows[t, pl.ds(c * _L, _L)])
                        for c in range(cl))
                return lax.fori_loop(0, _G, row_body, tuple(acc))

            @pl.when(npair > 0)
            def _():
                start(0, rows_a, sem_a)

            def pair_body(p, acc):
                j0 = 2 * p
                start(j0 + 1, rows_b, sem_b)
                wait(j0, rows_a, sem_a)
                acc = accum(rows_a, acc)

                @pl.when(j0 + 2 < nch)
                def _():
                    start(j0 + 2, rows_a, sem_a)

                wait(j0 + 1, rows_b, sem_b)
                return accum(rows_b, acc)

            acc0 = tuple(jnp.full((_L,), -jnp.inf, jnp.float32)
                         for _ in range(cl))
            acc = lax.fori_loop(0, npair, pair_body, acc0)
            nonempty = cnt > 0
            for c in range(cl):
                outrow_v[pl.ds(c * _L, _L)] = jnp.where(
                    nonempty, acc[c], jnp.float32(0.0))
            pltpu.sync_copy(outrow_v, out_hbm.at[g0 + r])
            return carry

        lax.fori_loop(0, rpw, roi_body, jnp.int32(0))

    mesh = plsc.VectorSubcoreMesh(core_axis_name="c", subcore_axis_name="s",
                                  num_cores=_NC, num_subcores=_NS)
    return pl.kernel(
        body,
        out_type=jax.ShapeDtypeStruct((B * K, C), jnp.float32),
        mesh=mesh,
        compiler_params=pltpu.CompilerParams(
            needs_layout_passes=False,
            use_tc_tiling_on_sc=False,
        ),
        scratch_types=[
            pltpu.VMEM((N,), jnp.float32),
            pltpu.VMEM((N,), jnp.float32),
            pltpu.VMEM((N,), jnp.float32),
            pltpu.VMEM((rpw * _L,), jnp.float32),
            pltpu.VMEM((N + 2 * _G,), jnp.int32),
            pltpu.VMEM((_G, C), jnp.float32),
            pltpu.VMEM((_G, C), jnp.float32),
            pltpu.VMEM((C,), jnp.float32),
            pltpu.SemaphoreType.DMA,
            pltpu.SemaphoreType.DMA,
        ],
    )


def _fc_body(p_ref, w1_ref, b1_ref, w2_ref, b2_ref, o_ref):
    h = jnp.dot(p_ref[...], w1_ref[...],
                preferred_element_type=jnp.float32) + b1_ref[...]
    h = jnp.maximum(h, 0.0)
    o = jnp.dot(h, w2_ref[...],
                preferred_element_type=jnp.float32) + b2_ref[...]
    o_ref[...] = jnp.maximum(o, 0.0)


def kernel(points, point_features, proposals, W1, b1, W2, b2):
    B, N, C = point_features.shape
    K = proposals.shape[1]

    xs = points[..., 0].reshape(B * N)
    ys = points[..., 1].reshape(B * N)
    zs = points[..., 2].reshape(B * N)
    feats_flat = point_features.reshape(B * N, C)
    ctr = proposals[..., 0:3]
    half = proposals[..., 3:6] / 2
    lo = ctr - half
    hi = ctr + half
    props = jnp.stack([lo[..., 0], hi[..., 0], lo[..., 1], hi[..., 1],
                       lo[..., 2], hi[..., 2]], axis=-1)
    props = jnp.concatenate(
        [props, jnp.zeros((B, K, _L - 6), jnp.float32)],
        axis=-1).reshape(B * K * _L)

    pooled = _make_pool(B, N, C, K)(xs, ys, zs, feats_flat, props)

    out = pl.pallas_call(
        _fc_body,
        out_shape=jax.ShapeDtypeStruct((B * K, W2.shape[1]), jnp.float32),
    )(pooled, W1, b1.reshape(1, -1), W2, b2.reshape(1, -1))
    return out.reshape(B, K, W2.shape[1])


# speedup vs baseline: 1.2725x; 1.1110x over previous
import jax
import jax.numpy as jnp
from jax import lax
from jax.experimental import pallas as pl
from jax.experimental.pallas import tpu as pltpu
from jax.experimental.pallas import tpu_sc as plsc

_NC = 2
_NS = 16
_NW = _NC * _NS
_L = 16
_G = 32
_NB = 4


def _make_pool(B, N, C, K):
    rpw = (B * K) // _NW
    cl = C // _L

    def body(xs_hbm, ys_hbm, zs_hbm, feats_hbm, props_hbm, out_hbm,
             xs_v, ys_v, zs_v, props_v, meml, rows_a, rows_b, rows_c,
             rows_d, outrow_v, sem_a, sem_b, sem_c, sem_d):
        wid = lax.axis_index("s") * _NC + lax.axis_index("c")
        g0 = wid * rpw
        base = (g0 // K) * N
        pltpu.sync_copy(xs_hbm.at[pl.ds(base, N)], xs_v)
        pltpu.sync_copy(ys_hbm.at[pl.ds(base, N)], ys_v)
        pltpu.sync_copy(zs_hbm.at[pl.ds(base, N)], zs_v)
        pltpu.sync_copy(props_hbm.at[pl.ds(g0 * _L, rpw * _L)], props_v)

        iota = lax.iota(jnp.int32, _L)

        def roi_body(r, carry):
            prow = props_v[pl.ds(r * _L, _L)]
            lox, hix, loy, hiy, loz, hiz = (prow[0], prow[1], prow[2],
                                            prow[3], prow[4], prow[5])

            def chunk(i, off):
                x = xs_v[pl.ds(i * _L, _L)]
                y = ys_v[pl.ds(i * _L, _L)]
                z = zs_v[pl.ds(i * _L, _L)]
                m = ((x > lox) & (x < hix) & (y > loy) & (y < hiy)
                     & (z > loz) & (z < hiz))
                idxv = (base + i * _L) + iota
                plsc.store_compressed(meml.at[pl.ds(off, _L)], idxv, mask=m)
                return off + plsc.all_reduce_population_count(m)[0]

            def mask_body(i2, off):
                off = chunk(2 * i2, off)
                return chunk(2 * i2 + 1, off)

            cnt = lax.fori_loop(0, N // (2 * _L), mask_body, jnp.int32(0))

            mv = meml[pl.ds(0, _L)]
            padv = mv[0] + (iota * 0)
            for q in range(_NB * _G // _L):
                plsc.store_scatter(meml, [cnt + q * _L + iota], padv)
            nquad = (cnt + (_NB * _G - 1)) // (_NB * _G)
            nch = _NB * nquad

            def start(j, rows, sem):
                pltpu.async_copy(
                    feats_hbm.at[meml.at[pl.ds(j * _G, _G)]], rows, sem)

            def wait(j, rows, sem):
                pltpu.make_async_copy(
                    feats_hbm.at[meml.at[pl.ds(j * _G, _G)]],
                    rows, sem).wait()

            def accum(rows, acc):
                def row_body(t, a):
                    return tuple(
                        jnp.maximum(a[c], rows[t, pl.ds(c * _L, _L)])
                        for c in range(cl))
                return lax.fori_loop(0, _G, row_body, tuple(acc))

            ring = ((rows_a, sem_a), (rows_b, sem_b),
                    (rows_c, sem_c), (rows_d, sem_d))

            @pl.when(nquad > 0)
            def _():
                for q in range(_NB - 1):
                    start(q, *ring[q])

            def quad_body(p, acc):
                j0 = _NB * p
                start(j0 + _NB - 1, *ring[_NB - 1])
                for q in range(_NB):
                    wait(j0 + q, *ring[q])
                    acc = accum(ring[q][0], acc)
                    if q < _NB - 1:
                        nxt = j0 + _NB + q

                        @pl.when(nxt < nch)
                        def _(q=q, nxt=nxt):
                            start(nxt, *ring[q])
                return acc

            acc0 = tuple(jnp.full((_L,), -jnp.inf, jnp.float32)
                         for _ in range(cl))
            acc = lax.fori_loop(0, nquad, quad_body, acc0)
            nonempty = cnt > 0
            for c in range(cl):
                outrow_v[pl.ds(c * _L, _L)] = jnp.where(
                    nonempty, acc[c], jnp.float32(0.0))
            pltpu.sync_copy(outrow_v, out_hbm.at[g0 + r])
            return carry

        lax.fori_loop(0, rpw, roi_body, jnp.int32(0))

    mesh = plsc.VectorSubcoreMesh(core_axis_name="c", subcore_axis_name="s",
                                  num_cores=_NC, num_subcores=_NS)
    return pl.kernel(
        body,
        out_type=jax.ShapeDtypeStruct((B * K, C), jnp.float32),
        mesh=mesh,
        compiler_params=pltpu.CompilerParams(
            needs_layout_passes=False,
            use_tc_tiling_on_sc=False,
        ),
        scratch_types=[
            pltpu.VMEM((N,), jnp.float32),
            pltpu.VMEM((N,), jnp.float32),
            pltpu.VMEM((N,), jnp.float32),
            pltpu.VMEM((rpw * _L,), jnp.float32),
            pltpu.VMEM((N + _NB * _G,), jnp.int32),
            pltpu.VMEM((_G, C), jnp.float32),
            pltpu.VMEM((_G, C), jnp.float32),
            pltpu.VMEM((_G, C), jnp.float32),
            pltpu.VMEM((_G, C), jnp.float32),
            pltpu.VMEM((C,), jnp.float32),
            pltpu.SemaphoreType.DMA,
            pltpu.SemaphoreType.DMA,
            pltpu.SemaphoreType.DMA,
            pltpu.SemaphoreType.DMA,
        ],
    )


def _fc_body(p_ref, w1_ref, b1_ref, w2_ref, b2_ref, o_ref):
    h = jnp.dot(p_ref[...], w1_ref[...],
                preferred_element_type=jnp.float32) + b1_ref[...]
    h = jnp.maximum(h, 0.0)
    o = jnp.dot(h, w2_ref[...],
                preferred_element_type=jnp.float32) + b2_ref[...]
    o_ref[...] = jnp.maximum(o, 0.0)


def kernel(points, point_features, proposals, W1, b1, W2, b2):
    B, N, C = point_features.shape
    K = proposals.shape[1]

    xs = points[..., 0].reshape(B * N)
    ys = points[..., 1].reshape(B * N)
    zs = points[..., 2].reshape(B * N)
    feats_flat = point_features.reshape(B * N, C)
    ctr = proposals[..., 0:3]
    half = proposals[..., 3:6] / 2
    lo = ctr - half
    hi = ctr + half
    props = jnp.stack([lo[..., 0], hi[..., 0], lo[..., 1], hi[..., 1],
                       lo[..., 2], hi[..., 2]], axis=-1)
    props = jnp.concatenate(
        [props, jnp.zeros((B, K, _L - 6), jnp.float32)],
        axis=-1).reshape(B * K * _L)

    pooled = _make_pool(B, N, C, K)(xs, ys, zs, feats_flat, props)

    out = pl.pallas_call(
        _fc_body,
        out_shape=jax.ShapeDtypeStruct((B * K, W2.shape[1]), jnp.float32),
    )(pooled, W1, b1.reshape(1, -1), W2, b2.reshape(1, -1))
    return out.reshape(B, K, W2.shape[1])


# speedup vs baseline: 1.3288x; 1.0442x over previous
import jax
import jax.numpy as jnp
from jax import lax
from jax.experimental import pallas as pl
from jax.experimental.pallas import tpu as pltpu
from jax.experimental.pallas import tpu_sc as plsc

_NC = 2
_NS = 16
_NW = _NC * _NS
_L = 16
_G = 32
_NB = 6


def _make_pool(B, N, C, K):
    rpw = (B * K) // _NW
    cl = C // _L

    def body(xs_hbm, ys_hbm, zs_hbm, feats_hbm, props_hbm, out_hbm,
             xs_v, ys_v, zs_v, props_v, meml, rows_a, rows_b, rows_c,
             rows_d, rows_e, rows_f, outrow_v,
             sem_a, sem_b, sem_c, sem_d, sem_e, sem_f):
        wid = lax.axis_index("s") * _NC + lax.axis_index("c")
        g0 = wid * rpw
        base = (g0 // K) * N
        pltpu.sync_copy(xs_hbm.at[pl.ds(base, N)], xs_v)
        pltpu.sync_copy(ys_hbm.at[pl.ds(base, N)], ys_v)
        pltpu.sync_copy(zs_hbm.at[pl.ds(base, N)], zs_v)
        pltpu.sync_copy(props_hbm.at[pl.ds(g0 * _L, rpw * _L)], props_v)

        iota = lax.iota(jnp.int32, _L)

        def roi_body(r, carry):
            prow = props_v[pl.ds(r * _L, _L)]
            lox, hix, loy, hiy, loz, hiz = (prow[0], prow[1], prow[2],
                                            prow[3], prow[4], prow[5])

            def chunk(i, off):
                x = xs_v[pl.ds(i * _L, _L)]
                y = ys_v[pl.ds(i * _L, _L)]
                z = zs_v[pl.ds(i * _L, _L)]
                m = ((x > lox) & (x < hix) & (y > loy) & (y < hiy)
                     & (z > loz) & (z < hiz))
                idxv = (base + i * _L) + iota
                plsc.store_compressed(meml.at[pl.ds(off, _L)], idxv, mask=m)
                return off + plsc.all_reduce_population_count(m)[0]

            def mask_body(i2, off):
                off = chunk(2 * i2, off)
                return chunk(2 * i2 + 1, off)

            cnt = lax.fori_loop(0, N // (2 * _L), mask_body, jnp.int32(0))

            mv = meml[pl.ds(0, _L)]
            padv = mv[0] + (iota * 0)
            for q in range(_NB * _G // _L):
                plsc.store_scatter(meml, [cnt + q * _L + iota], padv)
            nquad = (cnt + (_NB * _G - 1)) // (_NB * _G)
            nch = _NB * nquad

            def start(j, rows, sem):
                pltpu.async_copy(
                    feats_hbm.at[meml.at[pl.ds(j * _G, _G)]], rows, sem)

            def wait(j, rows, sem):
                pltpu.make_async_copy(
                    feats_hbm.at[meml.at[pl.ds(j * _G, _G)]],
                    rows, sem).wait()

            def accum(rows, acc):
                def row_body(t, a):
                    return tuple(
                        jnp.maximum(a[c], rows[t, pl.ds(c * _L, _L)])
                        for c in range(cl))
                return lax.fori_loop(0, _G, row_body, tuple(acc))

            ring = ((rows_a, sem_a), (rows_b, sem_b), (rows_c, sem_c),
                    (rows_d, sem_d), (rows_e, sem_e), (rows_f, sem_f))

            @pl.when(nquad > 0)
            def _():
                for q in range(_NB - 1):
                    start(q, *ring[q])

            def quad_body(p, acc):
                j0 = _NB * p
                start(j0 + _NB - 1, *ring[_NB - 1])
                for q in range(_NB):
                    wait(j0 + q, *ring[q])
                    acc = accum(ring[q][0], acc)
                    if q < _NB - 1:
                        nxt = j0 + _NB + q

                        @pl.when(nxt < nch)
                        def _(q=q, nxt=nxt):
                            start(nxt, *ring[q])
                return acc

            acc0 = tuple(jnp.full((_L,), -jnp.inf, jnp.float32)
                         for _ in range(cl))
            acc = lax.fori_loop(0, nquad, quad_body, acc0)
            nonempty = cnt > 0
            for c in range(cl):
                outrow_v[pl.ds(c * _L, _L)] = jnp.where(
                    nonempty, acc[c], jnp.float32(0.0))
            pltpu.sync_copy(outrow_v, out_hbm.at[g0 + r])
            return carry

        lax.fori_loop(0, rpw, roi_body, jnp.int32(0))

    mesh = plsc.VectorSubcoreMesh(core_axis_name="c", subcore_axis_name="s",
                                  num_cores=_NC, num_subcores=_NS)
    return pl.kernel(
        body,
        out_type=jax.ShapeDtypeStruct((B * K, C), jnp.float32),
        mesh=mesh,
        compiler_params=pltpu.CompilerParams(
            needs_layout_passes=False,
            use_tc_tiling_on_sc=False,
        ),
        scratch_types=[
            pltpu.VMEM((N,), jnp.float32),
            pltpu.VMEM((N,), jnp.float32),
            pltpu.VMEM((N,), jnp.float32),
            pltpu.VMEM((rpw * _L,), jnp.float32),
            pltpu.VMEM((N + _NB * _G,), jnp.int32),
            pltpu.VMEM((_G, C), jnp.float32),
            pltpu.VMEM((_G, C), jnp.float32),
            pltpu.VMEM((_G, C), jnp.float32),
            pltpu.VMEM((_G, C), jnp.float32),
            pltpu.VMEM((_G, C), jnp.float32),
            pltpu.VMEM((_G, C), jnp.float32),
            pltpu.VMEM((C,), jnp.float32),
            pltpu.SemaphoreType.DMA,
            pltpu.SemaphoreType.DMA,
            pltpu.SemaphoreType.DMA,
            pltpu.SemaphoreType.DMA,
            pltpu.SemaphoreType.DMA,
            pltpu.SemaphoreType.DMA,
        ],
    )


def _fc_body(p_ref, w1_ref, b1_ref, w2_ref, b2_ref, o_ref):
    h = jnp.dot(p_ref[...], w1_ref[...],
                preferred_element_type=jnp.float32) + b1_ref[...]
    h = jnp.maximum(h, 0.0)
    o = jnp.dot(h, w2_ref[...],
                preferred_element_type=jnp.float32) + b2_ref[...]
    o_ref[...] = jnp.maximum(o, 0.0)


def kernel(points, point_features, proposals, W1, b1, W2, b2):
    B, N, C = point_features.shape
    K = proposals.shape[1]

    xs = points[..., 0].reshape(B * N)
    ys = points[..., 1].reshape(B * N)
    zs = points[..., 2].reshape(B * N)
    feats_flat = point_features.reshape(B * N, C)
    ctr = proposals[..., 0:3]
    half = proposals[..., 3:6] / 2
    lo = ctr - half
    hi = ctr + half
    props = jnp.stack([lo[..., 0], hi[..., 0], lo[..., 1], hi[..., 1],
                       lo[..., 2], hi[..., 2]], axis=-1)
    props = jnp.concatenate(
        [props, jnp.zeros((B, K, _L - 6), jnp.float32)],
        axis=-1).reshape(B * K * _L)

    pooled = _make_pool(B, N, C, K)(xs, ys, zs, feats_flat, props)

    out = pl.pallas_call(
        _fc_body,
        out_shape=jax.ShapeDtypeStruct((B * K, W2.shape[1]), jnp.float32),
    )(pooled, W1, b1.reshape(1, -1), W2, b2.reshape(1, -1))
    return out.reshape(B, K, W2.shape[1])


# speedup vs baseline: 1.3458x; 1.0128x over previous
import jax
import jax.numpy as jnp
from jax import lax
from jax.experimental import pallas as pl
from jax.experimental.pallas import tpu as pltpu
from jax.experimental.pallas import tpu_sc as plsc

_NC = 2
_NS = 16
_NW = _NC * _NS
_L = 16
_G = 24
_NB = 8


def _make_pool(B, N, C, K):
    rpw = (B * K) // _NW
    cl = C // _L

    def body(xs_hbm, ys_hbm, zs_hbm, feats_hbm, props_hbm, out_hbm,
             xs_v, ys_v, zs_v, props_v, meml, rows_a, rows_b, rows_c,
             rows_d, rows_e, rows_f, rows_g, rows_h, outrow_v,
             sem_a, sem_b, sem_c, sem_d, sem_e, sem_f, sem_g, sem_h):
        wid = lax.axis_index("s") * _NC + lax.axis_index("c")
        g0 = wid * rpw
        base = (g0 // K) * N
        pltpu.sync_copy(xs_hbm.at[pl.ds(base, N)], xs_v)
        pltpu.sync_copy(ys_hbm.at[pl.ds(base, N)], ys_v)
        pltpu.sync_copy(zs_hbm.at[pl.ds(base, N)], zs_v)
        pltpu.sync_copy(props_hbm.at[pl.ds(g0 * _L, rpw * _L)], props_v)

        iota = lax.iota(jnp.int32, _L)

        def roi_body(r, carry):
            prow = props_v[pl.ds(r * _L, _L)]
            lox, hix, loy, hiy, loz, hiz = (prow[0], prow[1], prow[2],
                                            prow[3], prow[4], prow[5])

            def chunk(i, off):
                x = xs_v[pl.ds(i * _L, _L)]
                y = ys_v[pl.ds(i * _L, _L)]
                z = zs_v[pl.ds(i * _L, _L)]
                m = ((x > lox) & (x < hix) & (y > loy) & (y < hiy)
                     & (z > loz) & (z < hiz))
                idxv = (base + i * _L) + iota
                plsc.store_compressed(meml.at[pl.ds(off, _L)], idxv, mask=m)
                return off + plsc.all_reduce_population_count(m)[0]

            def mask_body(i2, off):
                off = chunk(2 * i2, off)
                return chunk(2 * i2 + 1, off)

            cnt = lax.fori_loop(0, N // (2 * _L), mask_body, jnp.int32(0))

            mv = meml[pl.ds(0, _L)]
            padv = mv[0] + (iota * 0)
            for q in range(_NB * _G // _L):
                plsc.store_scatter(meml, [cnt + q * _L + iota], padv)
            nquad = (cnt + (_NB * _G - 1)) // (_NB * _G)
            nch = _NB * nquad

            def start(j, rows, sem):
                pltpu.async_copy(
                    feats_hbm.at[meml.at[pl.ds(j * _G, _G)]], rows, sem)

            def wait(j, rows, sem):
                pltpu.make_async_copy(
                    feats_hbm.at[meml.at[pl.ds(j * _G, _G)]],
                    rows, sem).wait()

            def accum(rows, acc):
                def row_body(t, a):
                    return tuple(
                        jnp.maximum(a[c], rows[t, pl.ds(c * _L, _L)])
                        for c in range(cl))
                return lax.fori_loop(0, _G, row_body, tuple(acc))

            ring = ((rows_a, sem_a), (rows_b, sem_b), (rows_c, sem_c),
                    (rows_d, sem_d), (rows_e, sem_e), (rows_f, sem_f),
                    (rows_g, sem_g), (rows_h, sem_h))

            @pl.when(nquad > 0)
            def _():
                for q in range(_NB - 1):
                    start(q, *ring[q])

            def quad_body(p, acc):
                j0 = _NB * p
                start(j0 + _NB - 1, *ring[_NB - 1])
                for q in range(_NB):
                    wait(j0 + q, *ring[q])
                    acc = accum(ring[q][0], acc)
                    if q < _NB - 1:
                        nxt = j0 + _NB + q

                        @pl.when(nxt < nch)
                        def _(q=q, nxt=nxt):
                            start(nxt, *ring[q])
                return acc

            acc0 = tuple(jnp.full((_L,), -jnp.inf, jnp.float32)
                         for _ in range(cl))
            acc = lax.fori_loop(0, nquad, quad_body, acc0)
            nonempty = cnt > 0
            for c in range(cl):
                outrow_v[pl.ds(c * _L, _L)] = jnp.where(
                    nonempty, acc[c], jnp.float32(0.0))
            pltpu.sync_copy(outrow_v, out_hbm.at[g0 + r])
            return carry

        lax.fori_loop(0, rpw, roi_body, jnp.int32(0))

    mesh = plsc.VectorSubcoreMesh(core_axis_name="c", subcore_axis_name="s",
                                  num_cores=_NC, num_subcores=_NS)
    return pl.kernel(
        body,
        out_type=jax.ShapeDtypeStruct((B * K, C), jnp.float32),
        mesh=mesh,
        compiler_params=pltpu.CompilerParams(
            needs_layout_passes=False,
            use_tc_tiling_on_sc=False,
        ),
        scratch_types=[
            pltpu.VMEM((N,), jnp.float32),
            pltpu.VMEM((N,), jnp.float32),
            pltpu.VMEM((N,), jnp.float32),
            pltpu.VMEM((rpw * _L,), jnp.float32),
            pltpu.VMEM((N + _NB * _G,), jnp.int32),
            pltpu.VMEM((_G, C), jnp.float32),
            pltpu.VMEM((_G, C), jnp.float32),
            pltpu.VMEM((_G, C), jnp.float32),
            pltpu.VMEM((_G, C), jnp.float32),
            pltpu.VMEM((_G, C), jnp.float32),
            pltpu.VMEM((_G, C), jnp.float32),
            pltpu.VMEM((_G, C), jnp.float32),
            pltpu.VMEM((_G, C), jnp.float32),
            pltpu.VMEM((C,), jnp.float32),
            pltpu.SemaphoreType.DMA,
            pltpu.SemaphoreType.DMA,
            pltpu.SemaphoreType.DMA,
            pltpu.SemaphoreType.DMA,
            pltpu.SemaphoreType.DMA,
            pltpu.SemaphoreType.DMA,
            pltpu.SemaphoreType.DMA,
            pltpu.SemaphoreType.DMA,
        ],
    )


def _fc_body(p_ref, w1_ref, b1_ref, w2_ref, b2_ref, o_ref):
    h = jnp.dot(p_ref[...], w1_ref[...],
                preferred_element_type=jnp.float32) + b1_ref[...]
    h = jnp.maximum(h, 0.0)
    o = jnp.dot(h, w2_ref[...],
                preferred_element_type=jnp.float32) + b2_ref[...]
    o_ref[...] = jnp.maximum(o, 0.0)


def kernel(points, point_features, proposals, W1, b1, W2, b2):
    B, N, C = point_features.shape
    K = proposals.shape[1]

    xs = points[..., 0].reshape(B * N)
    ys = points[..., 1].reshape(B * N)
    zs = points[..., 2].reshape(B * N)
    feats_flat = point_features.reshape(B * N, C)
    ctr = proposals[..., 0:3]
    half = proposals[..., 3:6] / 2
    lo = ctr - half
    hi = ctr + half
    props = jnp.stack([lo[..., 0], hi[..., 0], lo[..., 1], hi[..., 1],
                       lo[..., 2], hi[..., 2]], axis=-1)
    props = jnp.concatenate(
        [props, jnp.zeros((B, K, _L - 6), jnp.float32)],
        axis=-1).reshape(B * K * _L)

    pooled = _make_pool(B, N, C, K)(xs, ys, zs, feats_flat, props)

    out = pl.pallas_call(
        _fc_body,
        out_shape=jax.ShapeDtypeStruct((B * K, W2.shape[1]), jnp.float32),
    )(pooled, W1, b1.reshape(1, -1), W2, b2.reshape(1, -1))
    return out.reshape(B, K, W2.shape[1])
